# Initial kernel scaffold; baseline (speedup 1.0000x reference)
import jax
import jax.numpy as jnp
from jax.experimental import pallas as pl


def kernel(padded_ids, W1, b1, W2, b2, gamma, beta):
    raise NotImplementedError("write your pallas kernel here")


# speedup vs baseline: 61.1421x; 61.1421x over previous
import functools

import jax
import jax.numpy as jnp
from jax.experimental import pallas as pl
from jax.experimental.pallas import tpu as pltpu

VOCAB_EXCL_PAD = 5
HIDDEN_DIM = 32
OUTPUT_DIM = 25
MAX_COUNT = 16.0
SEQ = 200

BR = 512


def _fused_body(ids_ref, w1t_ref, b1_ref, w2t_ref, b2_ref, gb_ref, out_ref):
    ids = ids_ref[...]
    ids = jnp.clip(ids, 0, VOCAB_EXCL_PAD)
    lane = jax.lax.broadcasted_iota(jnp.int32, ids.shape, 1)
    valid = lane < SEQ

    h = jnp.broadcast_to(b1_ref[0, :][None, :], (ids.shape[0], HIDDEN_DIM))
    for v in range(1, VOCAB_EXCL_PAD + 1):
        hit = jnp.where(valid & (ids == v), 1.0, 0.0)
        cnt = jnp.sum(hit, axis=1) * (1.0 / MAX_COUNT)
        h = h + cnt[:, None] * w1t_ref[v - 1, :][None, :]
    h = jnp.maximum(h, 0.0)

    h2 = jnp.dot(h, w2t_ref[...], preferred_element_type=jnp.float32)
    h2 = h2 + b2_ref[0, :][None, :]

    mean = jnp.mean(h2, axis=1, keepdims=True)
    d = h2 - mean
    var = jnp.mean(d * d, axis=1, keepdims=True)
    hn = d * jax.lax.rsqrt(var + 1e-5)
    hn = hn * gb_ref[0, :][None, :] + gb_ref[1, :][None, :]
    out_ref[...] = jnp.maximum(hn, 0.0)


@jax.jit
def kernel(padded_ids, W1, b1, W2, b2, gamma, beta):
    B = padded_ids.shape[0]
    ids = padded_ids.astype(jnp.int32)
    w1t = W1.T
    w2t = W2.T
    b1r = b1.reshape(1, HIDDEN_DIM)
    b2r = b2.reshape(1, OUTPUT_DIM)
    gb = jnp.stack([gamma, beta], axis=0)

    grid = (B // BR,)
    out = pl.pallas_call(
        _fused_body,
        grid=grid,
        in_specs=[
            pl.BlockSpec((BR, SEQ), lambda i: (i, 0)),
            pl.BlockSpec((VOCAB_EXCL_PAD, HIDDEN_DIM), lambda i: (0, 0)),
            pl.BlockSpec((1, HIDDEN_DIM), lambda i: (0, 0)),
            pl.BlockSpec((HIDDEN_DIM, OUTPUT_DIM), lambda i: (0, 0)),
            pl.BlockSpec((1, OUTPUT_DIM), lambda i: (0, 0)),
            pl.BlockSpec((2, OUTPUT_DIM), lambda i: (0, 0)),
        ],
        out_specs=pl.BlockSpec((BR, OUTPUT_DIM), lambda i: (i, 0)),
        out_shape=jax.ShapeDtypeStruct((B, OUTPUT_DIM), jnp.float32),
    )(ids, w1t, b1r, w2t, b2r, gb)
    return out
